# baseline (speedup 1.0000x reference)
import functools

import jax
import jax.numpy as jnp
from jax import lax
from jax.experimental import pallas as pl
from jax.experimental.pallas import tpu as pltpu
from jax.experimental.pallas import tpu_sc as plsc

B = 32
N = 100
C = 34
HW = 128 * 128
K = N * C
KP = 3456
NPAD = 128
CHUNK = 128
NCHUNK = KP // CHUNK
NGROUP = 3
GCHUNK = NCHUNK // NGROUP
GELEM = GCHUNK * CHUNK


@functools.partial(
    pl.kernel,
    out_type=jax.ShapeDtypeStruct((B, 128), jnp.float32),
    mesh=plsc.VectorSubcoreMesh(core_axis_name="c", subcore_axis_name="s"),
    compiler_params=pltpu.CompilerParams(needs_layout_passes=False),
    scratch_types=[
        pltpu.VMEM((NPAD,), jnp.int32),
        pltpu.VMEM((KP,), jnp.int32),
        pltpu.VMEM((KP,), jnp.int32),
        pltpu.VMEM((KP,), jnp.int32),
        pltpu.VMEM((KP,), jnp.float32),
        pltpu.VMEM((KP,), jnp.float32),
        pltpu.VMEM((KP,), jnp.float32),
        pltpu.VMEM((128,), jnp.float32),
        pltpu.SemaphoreType.DMA,
        pltpu.SemaphoreType.DMA,
        pltpu.SemaphoreType.DMA,
        pltpu.SemaphoreType.DMA,
        pltpu.SemaphoreType.DMA,
    ],
)
def _sc_loss(feat_hbm, ind_hbm, ntab_hbm, ctab_hbm, tgt_hbm, msk_hbm, out_hbm,
             ind_v, ntab_v, ctab_v, idx_v, pred_v, tgt_v, msk_v, out_v,
             sg0, sg1, sg2, sem_ind, sem_in):
    b = lax.axis_index("s") * 2 + lax.axis_index("c")
    lane = lax.iota(jnp.int32, 16)
    zf = jnp.zeros((16,), jnp.float32)
    groups = (sg0, sg1, sg2)

    pltpu.async_copy(ind_hbm.at[b], ind_v, sem_ind)
    pltpu.async_copy(ntab_hbm, ntab_v, sem_ind)
    pltpu.async_copy(ctab_hbm, ctab_v, sem_ind)
    pltpu.async_copy(tgt_hbm.at[b], tgt_v, sem_in)
    pltpu.async_copy(msk_hbm.at[b], msk_v, sem_in)
    pltpu.make_async_copy(ind_hbm.at[b], ind_v, sem_ind).wait()
    pltpu.make_async_copy(ntab_hbm, ntab_v, sem_ind).wait()
    pltpu.make_async_copy(ctab_hbm, ctab_v, sem_ind).wait()

    bbase = b * (C * HW)

    def make_build_fire(sem):
        def build_fire(m, _):
            coff = pl.multiple_of(m * CHUNK, CHUNK)
            for j in range(CHUNK // 16):
                o = coff + j * 16
                n = ntab_v[pl.ds(o, 16)]
                c = ctab_v[pl.ds(o, 16)]
                base = plsc.load_gather(ind_v, [n])
                idx_v[pl.ds(o, 16)] = (bbase + c) + base
            pltpu.async_copy(
                feat_hbm.at[idx_v.at[pl.ds(coff, CHUNK)]],
                pred_v.at[pl.ds(coff, CHUNK)],
                sem,
            )
            return 0
        return build_fire

    for g in range(NGROUP):
        lax.fori_loop(g * GCHUNK, (g + 1) * GCHUNK, make_build_fire(groups[g]), 0)

    pltpu.make_async_copy(tgt_hbm.at[b], tgt_v, sem_in).wait()
    pltpu.make_async_copy(msk_hbm.at[b], msk_v, sem_in).wait()

    def body(k, carry):
        accn, accd = carry
        o = pl.multiple_of(k * 16, 16)
        pv = pred_v[pl.ds(o, 16)]
        tv = tgt_v[pl.ds(o, 16)]
        mv = msk_v[pl.ds(o, 16)]
        accn = accn + jnp.abs(pv - tv) * mv
        accd = accd + mv
        return accn, accd

    accn, accd = zf, zf
    for g in range(NGROUP):
        pltpu.make_async_copy(
            feat_hbm.at[pl.ds(0, GELEM)],
            pred_v.at[pl.ds(g * GELEM, GELEM)],
            groups[g],
        ).wait()
        accn, accd = lax.fori_loop(
            g * (GELEM // 16), (g + 1) * (GELEM // 16), body, (accn, accd),
            unroll=4)

    out_v[pl.ds(0, 16)] = accn
    out_v[pl.ds(16, 16)] = accd
    pltpu.sync_copy(out_v, out_hbm.at[b])


def kernel(output, mask, ind, target):
    feat = output.reshape(-1)
    pos = jnp.arange(KP, dtype=jnp.int32)
    ntab = jnp.minimum(pos // C, N - 1)
    ctab = (pos - (pos // C) * C) * HW
    ind_p = jnp.pad(ind.astype(jnp.int32), ((0, 0), (0, NPAD - N)))
    tgt_p = jnp.pad(target.reshape(B, K), ((0, 0), (0, KP - K)))
    msk_p = jnp.pad(mask.reshape(B, K).astype(jnp.float32), ((0, 0), (0, KP - K)))
    parts = _sc_loss(feat, ind_p, ntab, ctab, tgt_p, msk_p)
    return jnp.sum(parts[:, 0:16]) / (jnp.sum(parts[:, 16:32]) + 0.0001)


# speedup vs baseline: 1.0855x; 1.0855x over previous
import functools

import jax
import jax.numpy as jnp
from jax import lax
from jax.experimental import pallas as pl
from jax.experimental.pallas import tpu as pltpu
from jax.experimental.pallas import tpu_sc as plsc

B = 32
N = 100
C = 34
HW = 128 * 128
K = N * C
KP = 3456
NPAD = 128
CHUNK = 128
NCHUNK = KP // CHUNK
NGROUP = 3
GCHUNK = NCHUNK // NGROUP
GELEM = GCHUNK * CHUNK


@functools.partial(
    pl.kernel,
    out_type=jax.ShapeDtypeStruct((B, 128), jnp.float32),
    mesh=plsc.VectorSubcoreMesh(core_axis_name="c", subcore_axis_name="s"),
    compiler_params=pltpu.CompilerParams(needs_layout_passes=False),
    scratch_types=[
        pltpu.VMEM((NPAD,), jnp.int32),
        pltpu.VMEM((KP,), jnp.int32),
        pltpu.VMEM((KP,), jnp.float32),
        pltpu.VMEM((KP,), jnp.float32),
        pltpu.VMEM((KP,), jnp.float32),
        pltpu.VMEM((128,), jnp.float32),
        pltpu.SemaphoreType.DMA,
        pltpu.SemaphoreType.DMA,
        pltpu.SemaphoreType.DMA,
        pltpu.SemaphoreType.DMA,
        pltpu.SemaphoreType.DMA,
    ],
)
def _sc_loss(feat_hbm, ind_hbm, tgt_hbm, msk_hbm, out_hbm,
             ind_v, idx_v, pred_v, tgt_v, msk_v, out_v,
             sg0, sg1, sg2, sem_ind, sem_in):
    b = lax.axis_index("s") * 2 + lax.axis_index("c")
    lane = lax.iota(jnp.int32, 16)
    zf = jnp.zeros((16,), jnp.float32)
    cvec = jnp.full((16,), C, jnp.int32)
    nmax = jnp.full((16,), N - 1, jnp.int32)
    groups = (sg0, sg1, sg2)

    pltpu.async_copy(ind_hbm.at[b], ind_v, sem_ind)
    pltpu.async_copy(tgt_hbm.at[b], tgt_v, sem_in)
    pltpu.async_copy(msk_hbm.at[b], msk_v, sem_in)
    pltpu.make_async_copy(ind_hbm.at[b], ind_v, sem_ind).wait()

    bbase = b * (C * HW)

    def make_build_fire(sem):
        def build_fire(m, _):
            coff = pl.multiple_of(m * CHUNK, CHUNK)
            for j in range(CHUNK // 16):
                p = lane + (coff + j * 16)
                n = lax.div(p, cvec)
                c = p - n * cvec
                n = jnp.minimum(n, nmax)
                base = plsc.load_gather(ind_v, [n])
                idx_v[pl.ds(coff + j * 16, 16)] = bbase + c * HW + base
            pltpu.async_copy(
                feat_hbm.at[idx_v.at[pl.ds(coff, CHUNK)]],
                pred_v.at[pl.ds(coff, CHUNK)],
                sem,
            )
            return 0
        return build_fire

    for g in range(NGROUP):
        lax.fori_loop(g * GCHUNK, (g + 1) * GCHUNK, make_build_fire(groups[g]), 0)

    pltpu.make_async_copy(tgt_hbm.at[b], tgt_v, sem_in).wait()
    pltpu.make_async_copy(msk_hbm.at[b], msk_v, sem_in).wait()

    def body(k, carry):
        accn, accd = carry
        o = pl.multiple_of(k * 16, 16)
        pv = pred_v[pl.ds(o, 16)]
        tv = tgt_v[pl.ds(o, 16)]
        mv = msk_v[pl.ds(o, 16)]
        accn = accn + jnp.abs(pv - tv) * mv
        accd = accd + mv
        return accn, accd

    accn, accd = zf, zf
    for g in range(NGROUP):
        pltpu.make_async_copy(
            feat_hbm.at[pl.ds(0, GELEM)],
            pred_v.at[pl.ds(g * GELEM, GELEM)],
            groups[g],
        ).wait()
        accn, accd = lax.fori_loop(
            g * (GELEM // 16), (g + 1) * (GELEM // 16), body, (accn, accd),
            unroll=4)

    out_v[pl.ds(0, 16)] = accn
    out_v[pl.ds(16, 16)] = accd
    pltpu.sync_copy(out_v, out_hbm.at[b])


def kernel(output, mask, ind, target):
    feat = output.reshape(-1)
    ind_p = jnp.pad(ind.astype(jnp.int32), ((0, 0), (0, NPAD - N)))
    tgt_p = jnp.pad(target.reshape(B, K), ((0, 0), (0, KP - K)))
    msk_p = jnp.pad(mask.reshape(B, K).astype(jnp.float32), ((0, 0), (0, KP - K)))
    parts = _sc_loss(feat, ind_p, tgt_p, msk_p)
    return jnp.sum(parts[:, 0:16]) / (jnp.sum(parts[:, 16:32]) + 0.0001)
